# baseline (speedup 1.0000x reference)
import functools

import jax
import jax.numpy as jnp
from jax import lax
from jax.experimental import pallas as pl
from jax.experimental.pallas import tpu as pltpu
from jax.experimental.pallas import tpu_sc as plsc

N = 10000
C = 128
E = 320000
T = 4

_NC = 2
_NS = 16
_NP = 10240
_E2 = 2 * E
_EDGE_CHUNK = 32
_EDGES_PER_TILE = _E2 // (_NC * _NS)
_CHUNKS_PER_TILE = _EDGES_PER_TILE // _EDGE_CHUNK
_ROWS_PER_TILE = _NP // _NS


_NROW = 5
_GD = 3
_SD = 2
_NIDX = 10
_IA = 5
_UNROLL = 10


def _segsum_sc(u_all, src_all, dst_all, zrows):
    mesh = plsc.VectorSubcoreMesh(core_axis_name="c", subcore_axis_name="s")
    CH = _CHUNKS_PER_TILE
    CHM = CH - CH % _UNROLL

    @functools.partial(
        pl.kernel,
        mesh=mesh,
        out_type=jax.ShapeDtypeStruct((2 * _NP, C), jnp.float32),
        scratch_types=(
            [pltpu.VMEM((_EDGE_CHUNK,), jnp.int32) for _ in range(2 * _NIDX)]
            + [pltpu.VMEM((_EDGE_CHUNK, C), jnp.float32) for _ in range(_NROW)]
            + [pltpu.SemaphoreType.DMA for _ in range(_NIDX + 2 * _NROW)]
            + [pltpu.VMEM_SHARED((_NP, C), jnp.float32)]
        ),
    )
    def k(u_hbm, src_hbm, dst_hbm, z_hbm, out_hbm, *scr):
        idxs = scr[:_NIDX]
        idxd = scr[_NIDX:2 * _NIDX]
        rows = scr[2 * _NIDX:2 * _NIDX + _NROW]
        isem = scr[2 * _NIDX + _NROW:2 * _NIDX + _NROW + _NIDX]
        gsem = scr[3 * _NIDX + _NROW:3 * _NIDX + _NROW + _NROW]
        ssem = scr[3 * _NIDX + 2 * _NROW:3 * _NIDX + 3 * _NROW]
        agg = scr[3 * _NIDX + 3 * _NROW]
        cid = lax.axis_index("c")
        sid = lax.axis_index("s")
        wid = cid * _NS + sid
        r0 = sid * _ROWS_PER_TILE
        base = wid * _EDGES_PER_TILE
        pltpu.sync_copy(z_hbm, agg.at[pl.ds(r0, _ROWS_PER_TILE)])
        plsc.subcore_barrier()

        def issue_i(j, b8):
            off = base + j * _EDGE_CHUNK
            pltpu.async_copy(src_hbm.at[pl.ds(off, _EDGE_CHUNK)], idxs[b8], isem[b8])
            pltpu.async_copy(dst_hbm.at[pl.ds(off, _EDGE_CHUNK)], idxd[b8], isem[b8])

        def wait_i(j, b8):
            off = base + j * _EDGE_CHUNK
            pltpu.make_async_copy(src_hbm.at[pl.ds(off, _EDGE_CHUNK)], idxs[b8], isem[b8]).wait()
            pltpu.make_async_copy(dst_hbm.at[pl.ds(off, _EDGE_CHUNK)], idxd[b8], isem[b8]).wait()

        def issue_g(b8, b4):
            pltpu.async_copy(u_hbm.at[idxs[b8]], rows[b4], gsem[b4])

        def wait_g(b8, b4):
            pltpu.make_async_copy(u_hbm.at[idxs[b8]], rows[b4], gsem[b4]).wait()

        def issue_s(b8, b4):
            pltpu.async_copy(rows[b4], agg.at[idxd[b8]], ssem[b4], add=True)

        def wait_s(b8, b4):
            pltpu.make_async_copy(rows[b4], agg.at[idxd[b8]], ssem[b4]).wait()

        for j0 in range(_IA):
            issue_i(j0, j0 % _NIDX)
        for j0 in range(_GD):
            wait_i(j0, j0 % _NIDX)
            issue_g(j0 % _NIDX, j0 % _NROW)

        def body(jo, carry):
            jb = jo * _UNROLL
            for b in range(_UNROLL):
                j = jb + b
                wait_g(b % _NIDX, b % _NROW)
                issue_s(b % _NIDX, b % _NROW)

                @pl.when(j - _SD >= 0)
                def _(b=b):
                    wait_s((b - _SD) % _NIDX, (b - _SD) % _NROW)

                @pl.when(j + _IA < CHM)
                def _(j=j, b=b):
                    issue_i(j + _IA, (b + _IA) % _NIDX)

                @pl.when(j + _GD < CHM)
                def _(j=j, b=b):
                    wait_i(j + _GD, (b + _GD) % _NIDX)
                    issue_g((b + _GD) % _NIDX, (b + _GD) % _NROW)

            return carry

        lax.fori_loop(0, CHM // _UNROLL, body, 0)
        for jt in range(CHM - _SD, CHM):
            wait_s(jt % _NIDX, jt % _NROW)
        for jt in range(CHM, CH):
            b = jt % _NIDX
            issue_i(jt, b)
            wait_i(jt, b)
            issue_g(b, jt % _NROW)
            wait_g(b, jt % _NROW)
            issue_s(b, jt % _NROW)
            wait_s(b, jt % _NROW)
        plsc.subcore_barrier()
        pltpu.sync_copy(
            agg.at[pl.ds(r0, _ROWS_PER_TILE)],
            out_hbm.at[pl.ds(cid * _NP + r0, _ROWS_PER_TILE)],
        )

    return k(u_all, src_all, dst_all, zrows)


_BLK = 1024


def _mlp_tc(agg, x_all, w1t, b1, s1t, c1, s2t, c2, s3t, c3):
    M = 2 * _NP

    def body(agg_ref, x_ref, w1_ref, b1_ref, s1_ref, c1_ref, s2_ref, c2_ref,
             s3_ref, c3_ref, out_ref):
        f32 = jnp.float32
        h = jnp.dot(x_ref[...], w1_ref[...], preferred_element_type=f32) + b1_ref[...]
        s = jnp.dot(agg_ref[...], s1_ref[...], preferred_element_type=f32) + c1_ref[...]
        s = jnp.maximum(s, 0.0)
        s = jnp.dot(s, s2_ref[...], preferred_element_type=f32) + c2_ref[...]
        s = jnp.maximum(s, 0.0)
        s = jnp.tanh(jnp.dot(s, s3_ref[...], preferred_element_type=f32) + c3_ref[...])
        out_ref[...] = jnp.maximum(h + s, 0.0)

    row = lambda i: (i, 0)
    full = lambda i: (0, 0)
    return pl.pallas_call(
        body,
        grid=(M // _BLK,),
        in_specs=[
            pl.BlockSpec((_BLK, C), row),
            pl.BlockSpec((_BLK, C), row),
            pl.BlockSpec((C, C), full),
            pl.BlockSpec((1, C), full),
            pl.BlockSpec((C, C), full),
            pl.BlockSpec((1, C), full),
            pl.BlockSpec((C, C), full),
            pl.BlockSpec((1, C), full),
            pl.BlockSpec((C, C), full),
            pl.BlockSpec((1, C), full),
        ],
        out_specs=pl.BlockSpec((_BLK, C), row),
        out_shape=jax.ShapeDtypeStruct((M, C), jnp.float32),
    )(agg, x_all, w1t, b1, s1t, c1, s2t, c2, s3t, c3)


def _final_tc(u_all, w2t, b2):
    def body(u_ref, w_ref, b_ref, out_ref):
        u = u_ref[...]
        g1 = jnp.sum(u[0:N], axis=0, keepdims=True)
        g2 = jnp.sum(u[_NP:_NP + N], axis=0, keepdims=True)
        g = jnp.concatenate([g1, g2], axis=0)
        out_ref[...] = jnp.dot(g, w_ref[...], preferred_element_type=jnp.float32) + b_ref[...]

    return pl.pallas_call(
        body,
        out_shape=jax.ShapeDtypeStruct((2, C), jnp.float32),
    )(u_all, w2t, b2)


def kernel(x1, u1, edge_index1, x2, u2, edge_index2, params):
    pad = jnp.zeros((_NP - N, C), jnp.float32)
    src_all = jnp.concatenate([edge_index1[0], edge_index2[0] + _NP])
    dst_all = jnp.concatenate([edge_index1[1], edge_index2[1]])
    u_all = jnp.concatenate([u1, pad, u2, pad], axis=0)
    x_all = jnp.concatenate([x1, pad, x2, pad], axis=0)
    zrows = jnp.zeros((_ROWS_PER_TILE, C), jnp.float32)

    for t in range(T):
        p = params["convs"][t]
        agg = _segsum_sc(u_all, src_all, dst_all, zrows)
        u_all = _mlp_tc(
            agg, x_all,
            p["w1"][0].T, p["w1"][1].reshape(1, C),
            p["s1"][0].T, p["s1"][1].reshape(1, C),
            p["s2"][0].T, p["s2"][1].reshape(1, C),
            p["s3"][0].T, p["s3"][1].reshape(1, C),
        )

    g = _final_tc(u_all, params["w2"][0].T, params["w2"][1].reshape(1, C))
    return (g[0:1], g[1:2])


# speedup vs baseline: 1.1484x; 1.1484x over previous
import functools

import jax
import jax.numpy as jnp
from jax import lax
from jax.experimental import pallas as pl
from jax.experimental.pallas import tpu as pltpu
from jax.experimental.pallas import tpu_sc as plsc

N = 10000
C = 128
E = 320000
T = 4

_NC = 2
_NS = 16
_NP = 10240
_E2 = 2 * E
_EDGE_CHUNK = 40
_EDGES_PER_TILE = _E2 // (_NC * _NS)
_CHUNKS_PER_TILE = _EDGES_PER_TILE // _EDGE_CHUNK
_ROWS_PER_TILE = _NP // _NS


_NROW = 6
_GD = 4
_SD = 2
_NIDX = 12
_IA = 6
_UNROLL = 12


def _segsum_sc(u_all, src_all, dst_all, zrows):
    mesh = plsc.VectorSubcoreMesh(core_axis_name="c", subcore_axis_name="s")
    CH = _CHUNKS_PER_TILE
    CHM = CH - CH % _UNROLL

    @functools.partial(
        pl.kernel,
        mesh=mesh,
        out_type=jax.ShapeDtypeStruct((2 * _NP, C), jnp.float32),
        scratch_types=(
            [pltpu.VMEM((_EDGE_CHUNK,), jnp.int32) for _ in range(2 * _NIDX)]
            + [pltpu.VMEM((_EDGE_CHUNK, C), jnp.float32) for _ in range(_NROW)]
            + [pltpu.SemaphoreType.DMA for _ in range(_NIDX + 2 * _NROW)]
            + [pltpu.VMEM_SHARED((_NP, C), jnp.float32)]
        ),
    )
    def k(u_hbm, src_hbm, dst_hbm, z_hbm, out_hbm, *scr):
        idxs = scr[:_NIDX]
        idxd = scr[_NIDX:2 * _NIDX]
        rows = scr[2 * _NIDX:2 * _NIDX + _NROW]
        isem = scr[2 * _NIDX + _NROW:2 * _NIDX + _NROW + _NIDX]
        gsem = scr[3 * _NIDX + _NROW:3 * _NIDX + _NROW + _NROW]
        ssem = scr[3 * _NIDX + 2 * _NROW:3 * _NIDX + 3 * _NROW]
        agg = scr[3 * _NIDX + 3 * _NROW]
        cid = lax.axis_index("c")
        sid = lax.axis_index("s")
        wid = cid * _NS + sid
        r0 = sid * _ROWS_PER_TILE
        base = wid * _EDGES_PER_TILE
        pltpu.sync_copy(z_hbm, agg.at[pl.ds(r0, _ROWS_PER_TILE)])
        plsc.subcore_barrier()

        def issue_i(j, b8):
            off = base + j * _EDGE_CHUNK
            pltpu.async_copy(src_hbm.at[pl.ds(off, _EDGE_CHUNK)], idxs[b8], isem[b8])
            pltpu.async_copy(dst_hbm.at[pl.ds(off, _EDGE_CHUNK)], idxd[b8], isem[b8])

        def wait_i(j, b8):
            off = base + j * _EDGE_CHUNK
            pltpu.make_async_copy(src_hbm.at[pl.ds(off, _EDGE_CHUNK)], idxs[b8], isem[b8]).wait()
            pltpu.make_async_copy(dst_hbm.at[pl.ds(off, _EDGE_CHUNK)], idxd[b8], isem[b8]).wait()

        def issue_g(b8, b4):
            pltpu.async_copy(u_hbm.at[idxs[b8]], rows[b4], gsem[b4])

        def wait_g(b8, b4):
            pltpu.make_async_copy(u_hbm.at[idxs[b8]], rows[b4], gsem[b4]).wait()

        def issue_s(b8, b4):
            pltpu.async_copy(rows[b4], agg.at[idxd[b8]], ssem[b4], add=True)

        def wait_s(b8, b4):
            pltpu.make_async_copy(rows[b4], agg.at[idxd[b8]], ssem[b4]).wait()

        for j0 in range(_IA):
            issue_i(j0, j0 % _NIDX)
        for j0 in range(_GD):
            wait_i(j0, j0 % _NIDX)
            issue_g(j0 % _NIDX, j0 % _NROW)

        def body(jo, carry):
            jb = jo * _UNROLL
            for b in range(_UNROLL):
                j = jb + b
                wait_g(b % _NIDX, b % _NROW)
                issue_s(b % _NIDX, b % _NROW)

                @pl.when(j - _SD >= 0)
                def _(b=b):
                    wait_s((b - _SD) % _NIDX, (b - _SD) % _NROW)

                @pl.when(j + _IA < CHM)
                def _(j=j, b=b):
                    issue_i(j + _IA, (b + _IA) % _NIDX)

                @pl.when(j + _GD < CHM)
                def _(j=j, b=b):
                    wait_i(j + _GD, (b + _GD) % _NIDX)
                    issue_g((b + _GD) % _NIDX, (b + _GD) % _NROW)

            return carry

        lax.fori_loop(0, CHM // _UNROLL, body, 0)
        for jt in range(CHM - _SD, CHM):
            wait_s(jt % _NIDX, jt % _NROW)
        for jt in range(CHM, CH):
            b = jt % _NIDX
            issue_i(jt, b)
            wait_i(jt, b)
            issue_g(b, jt % _NROW)
            wait_g(b, jt % _NROW)
            issue_s(b, jt % _NROW)
            wait_s(b, jt % _NROW)
        plsc.subcore_barrier()
        pltpu.sync_copy(
            agg.at[pl.ds(r0, _ROWS_PER_TILE)],
            out_hbm.at[pl.ds(cid * _NP + r0, _ROWS_PER_TILE)],
        )

    return k(u_all, src_all, dst_all, zrows)


_BLK = 1024


def _mlp_tc(agg, x_all, w1t, b1, s1t, c1, s2t, c2, s3t, c3):
    M = 2 * _NP

    def body(agg_ref, x_ref, w1_ref, b1_ref, s1_ref, c1_ref, s2_ref, c2_ref,
             s3_ref, c3_ref, out_ref):
        f32 = jnp.float32
        h = jnp.dot(x_ref[...], w1_ref[...], preferred_element_type=f32) + b1_ref[...]
        s = jnp.dot(agg_ref[...], s1_ref[...], preferred_element_type=f32) + c1_ref[...]
        s = jnp.maximum(s, 0.0)
        s = jnp.dot(s, s2_ref[...], preferred_element_type=f32) + c2_ref[...]
        s = jnp.maximum(s, 0.0)
        s = jnp.tanh(jnp.dot(s, s3_ref[...], preferred_element_type=f32) + c3_ref[...])
        out_ref[...] = jnp.maximum(h + s, 0.0)

    row = lambda i: (i, 0)
    full = lambda i: (0, 0)
    return pl.pallas_call(
        body,
        grid=(M // _BLK,),
        in_specs=[
            pl.BlockSpec((_BLK, C), row),
            pl.BlockSpec((_BLK, C), row),
            pl.BlockSpec((C, C), full),
            pl.BlockSpec((1, C), full),
            pl.BlockSpec((C, C), full),
            pl.BlockSpec((1, C), full),
            pl.BlockSpec((C, C), full),
            pl.BlockSpec((1, C), full),
            pl.BlockSpec((C, C), full),
            pl.BlockSpec((1, C), full),
        ],
        out_specs=pl.BlockSpec((_BLK, C), row),
        out_shape=jax.ShapeDtypeStruct((M, C), jnp.float32),
    )(agg, x_all, w1t, b1, s1t, c1, s2t, c2, s3t, c3)


def _final_tc(u_all, w2t, b2):
    def body(u_ref, w_ref, b_ref, out_ref):
        u = u_ref[...]
        g1 = jnp.sum(u[0:N], axis=0, keepdims=True)
        g2 = jnp.sum(u[_NP:_NP + N], axis=0, keepdims=True)
        g = jnp.concatenate([g1, g2], axis=0)
        out_ref[...] = jnp.dot(g, w_ref[...], preferred_element_type=jnp.float32) + b_ref[...]

    return pl.pallas_call(
        body,
        out_shape=jax.ShapeDtypeStruct((2, C), jnp.float32),
    )(u_all, w2t, b2)


def kernel(x1, u1, edge_index1, x2, u2, edge_index2, params):
    pad = jnp.zeros((_NP - N, C), jnp.float32)
    src_all = jnp.concatenate([edge_index1[0], edge_index2[0] + _NP])
    dst_all = jnp.concatenate([edge_index1[1], edge_index2[1]])
    u_all = jnp.concatenate([u1, pad, u2, pad], axis=0)
    x_all = jnp.concatenate([x1, pad, x2, pad], axis=0)
    zrows = jnp.zeros((_ROWS_PER_TILE, C), jnp.float32)

    for t in range(T):
        p = params["convs"][t]
        agg = _segsum_sc(u_all, src_all, dst_all, zrows)
        u_all = _mlp_tc(
            agg, x_all,
            p["w1"][0].T, p["w1"][1].reshape(1, C),
            p["s1"][0].T, p["s1"][1].reshape(1, C),
            p["s2"][0].T, p["s2"][1].reshape(1, C),
            p["s3"][0].T, p["s3"][1].reshape(1, C),
        )

    g = _final_tc(u_all, params["w2"][0].T, params["w2"][1].reshape(1, C))
    return (g[0:1], g[1:2])


# speedup vs baseline: 1.1514x; 1.0025x over previous
import functools

import jax
import jax.numpy as jnp
from jax import lax
from jax.experimental import pallas as pl
from jax.experimental.pallas import tpu as pltpu
from jax.experimental.pallas import tpu_sc as plsc

N = 10000
C = 128
E = 320000
T = 4

_NC = 2
_NS = 16
_NP = 10240
_E2 = 2 * E
_EDGE_CHUNK = 40
_EDGES_PER_TILE = _E2 // (_NC * _NS)
_CHUNKS_PER_TILE = _EDGES_PER_TILE // _EDGE_CHUNK
_ROWS_PER_TILE = _NP // _NS


_NROW = 7
_GD = 5
_SD = 2
_NIDX = 14
_IA = 7
_UNROLL = 14


def _segsum_sc(u_all, src_all, dst_all, zrows):
    mesh = plsc.VectorSubcoreMesh(core_axis_name="c", subcore_axis_name="s")
    CH = _CHUNKS_PER_TILE
    CHM = CH - CH % _UNROLL

    @functools.partial(
        pl.kernel,
        mesh=mesh,
        out_type=jax.ShapeDtypeStruct((2 * _NP, C), jnp.float32),
        scratch_types=(
            [pltpu.VMEM((_EDGE_CHUNK,), jnp.int32) for _ in range(2 * _NIDX)]
            + [pltpu.VMEM((_EDGE_CHUNK, C), jnp.float32) for _ in range(_NROW)]
            + [pltpu.SemaphoreType.DMA for _ in range(_NIDX + 2 * _NROW)]
            + [pltpu.VMEM_SHARED((_NP, C), jnp.float32)]
        ),
    )
    def k(u_hbm, src_hbm, dst_hbm, z_hbm, out_hbm, *scr):
        idxs = scr[:_NIDX]
        idxd = scr[_NIDX:2 * _NIDX]
        rows = scr[2 * _NIDX:2 * _NIDX + _NROW]
        isem = scr[2 * _NIDX + _NROW:2 * _NIDX + _NROW + _NIDX]
        gsem = scr[3 * _NIDX + _NROW:3 * _NIDX + _NROW + _NROW]
        ssem = scr[3 * _NIDX + 2 * _NROW:3 * _NIDX + 3 * _NROW]
        agg = scr[3 * _NIDX + 3 * _NROW]
        cid = lax.axis_index("c")
        sid = lax.axis_index("s")
        wid = cid * _NS + sid
        r0 = sid * _ROWS_PER_TILE
        base = wid * _EDGES_PER_TILE
        pltpu.sync_copy(z_hbm, agg.at[pl.ds(r0, _ROWS_PER_TILE)])
        plsc.subcore_barrier()

        def issue_i(j, b8):
            off = base + j * _EDGE_CHUNK
            pltpu.async_copy(src_hbm.at[pl.ds(off, _EDGE_CHUNK)], idxs[b8], isem[b8])
            pltpu.async_copy(dst_hbm.at[pl.ds(off, _EDGE_CHUNK)], idxd[b8], isem[b8])

        def wait_i(j, b8):
            off = base + j * _EDGE_CHUNK
            pltpu.make_async_copy(src_hbm.at[pl.ds(off, _EDGE_CHUNK)], idxs[b8], isem[b8]).wait()
            pltpu.make_async_copy(dst_hbm.at[pl.ds(off, _EDGE_CHUNK)], idxd[b8], isem[b8]).wait()

        def issue_g(b8, b4):
            pltpu.async_copy(u_hbm.at[idxs[b8]], rows[b4], gsem[b4])

        def wait_g(b8, b4):
            pltpu.make_async_copy(u_hbm.at[idxs[b8]], rows[b4], gsem[b4]).wait()

        def issue_s(b8, b4):
            pltpu.async_copy(rows[b4], agg.at[idxd[b8]], ssem[b4], add=True)

        def wait_s(b8, b4):
            pltpu.make_async_copy(rows[b4], agg.at[idxd[b8]], ssem[b4]).wait()

        for j0 in range(_IA):
            issue_i(j0, j0 % _NIDX)
        for j0 in range(_GD):
            wait_i(j0, j0 % _NIDX)
            issue_g(j0 % _NIDX, j0 % _NROW)

        def body(jo, carry):
            jb = jo * _UNROLL
            for b in range(_UNROLL):
                j = jb + b
                wait_g(b % _NIDX, b % _NROW)
                issue_s(b % _NIDX, b % _NROW)

                @pl.when(j - _SD >= 0)
                def _(b=b):
                    wait_s((b - _SD) % _NIDX, (b - _SD) % _NROW)

                @pl.when(j + _IA < CHM)
                def _(j=j, b=b):
                    issue_i(j + _IA, (b + _IA) % _NIDX)

                @pl.when(j + _GD < CHM)
                def _(j=j, b=b):
                    wait_i(j + _GD, (b + _GD) % _NIDX)
                    issue_g((b + _GD) % _NIDX, (b + _GD) % _NROW)

            return carry

        lax.fori_loop(0, CHM // _UNROLL, body, 0)
        for jt in range(CHM - _SD, CHM):
            wait_s(jt % _NIDX, jt % _NROW)
        for jt in range(CHM, CH):
            b = jt % _NIDX
            issue_i(jt, b)
            wait_i(jt, b)
            issue_g(b, jt % _NROW)
            wait_g(b, jt % _NROW)
            issue_s(b, jt % _NROW)
            wait_s(b, jt % _NROW)
        plsc.subcore_barrier()
        pltpu.sync_copy(
            agg.at[pl.ds(r0, _ROWS_PER_TILE)],
            out_hbm.at[pl.ds(cid * _NP + r0, _ROWS_PER_TILE)],
        )

    return k(u_all, src_all, dst_all, zrows)


_BLK = 1024


def _mlp_tc(agg, x_all, w1t, b1, s1t, c1, s2t, c2, s3t, c3):
    M = 2 * _NP

    def body(agg_ref, x_ref, w1_ref, b1_ref, s1_ref, c1_ref, s2_ref, c2_ref,
             s3_ref, c3_ref, out_ref):
        f32 = jnp.float32
        h = jnp.dot(x_ref[...], w1_ref[...], preferred_element_type=f32) + b1_ref[...]
        s = jnp.dot(agg_ref[...], s1_ref[...], preferred_element_type=f32) + c1_ref[...]
        s = jnp.maximum(s, 0.0)
        s = jnp.dot(s, s2_ref[...], preferred_element_type=f32) + c2_ref[...]
        s = jnp.maximum(s, 0.0)
        s = jnp.tanh(jnp.dot(s, s3_ref[...], preferred_element_type=f32) + c3_ref[...])
        out_ref[...] = jnp.maximum(h + s, 0.0)

    row = lambda i: (i, 0)
    full = lambda i: (0, 0)
    return pl.pallas_call(
        body,
        grid=(M // _BLK,),
        in_specs=[
            pl.BlockSpec((_BLK, C), row),
            pl.BlockSpec((_BLK, C), row),
            pl.BlockSpec((C, C), full),
            pl.BlockSpec((1, C), full),
            pl.BlockSpec((C, C), full),
            pl.BlockSpec((1, C), full),
            pl.BlockSpec((C, C), full),
            pl.BlockSpec((1, C), full),
            pl.BlockSpec((C, C), full),
            pl.BlockSpec((1, C), full),
        ],
        out_specs=pl.BlockSpec((_BLK, C), row),
        out_shape=jax.ShapeDtypeStruct((M, C), jnp.float32),
    )(agg, x_all, w1t, b1, s1t, c1, s2t, c2, s3t, c3)


def _final_tc(u_all, w2t, b2):
    def body(u_ref, w_ref, b_ref, out_ref):
        u = u_ref[...]
        g1 = jnp.sum(u[0:N], axis=0, keepdims=True)
        g2 = jnp.sum(u[_NP:_NP + N], axis=0, keepdims=True)
        g = jnp.concatenate([g1, g2], axis=0)
        out_ref[...] = jnp.dot(g, w_ref[...], preferred_element_type=jnp.float32) + b_ref[...]

    return pl.pallas_call(
        body,
        out_shape=jax.ShapeDtypeStruct((2, C), jnp.float32),
    )(u_all, w2t, b2)


def kernel(x1, u1, edge_index1, x2, u2, edge_index2, params):
    pad = jnp.zeros((_NP - N, C), jnp.float32)
    src_all = jnp.concatenate([edge_index1[0], edge_index2[0] + _NP])
    dst_all = jnp.concatenate([edge_index1[1], edge_index2[1]])
    u_all = jnp.concatenate([u1, pad, u2, pad], axis=0)
    x_all = jnp.concatenate([x1, pad, x2, pad], axis=0)
    zrows = jnp.zeros((_ROWS_PER_TILE, C), jnp.float32)

    for t in range(T):
        p = params["convs"][t]
        agg = _segsum_sc(u_all, src_all, dst_all, zrows)
        u_all = _mlp_tc(
            agg, x_all,
            p["w1"][0].T, p["w1"][1].reshape(1, C),
            p["s1"][0].T, p["s1"][1].reshape(1, C),
            p["s2"][0].T, p["s2"][1].reshape(1, C),
            p["s3"][0].T, p["s3"][1].reshape(1, C),
        )

    g = _final_tc(u_all, params["w2"][0].T, params["w2"][1].reshape(1, C))
    return (g[0:1], g[1:2])
